# baseline (speedup 1.0000x reference)
import functools

import jax
import jax.numpy as jnp
from jax import lax
from jax.experimental import pallas as pl
from jax.experimental.pallas import tpu as pltpu
from jax.experimental.pallas import tpu_sc as plsc

D = 32
NF = 26
BATCH = 16384
NC, NS = 2, 16
NW = NC * NS
BW = BATCH // NW
CH = 128
NCHUNK = NF * (BW // CH)
PITCH = 129
IPITCH = 520

_mesh = plsc.VectorSubcoreMesh(core_axis_name="c", subcore_axis_name="s")


@functools.partial(
    pl.kernel,
    mesh=_mesh,
    out_type=jax.ShapeDtypeStruct((NF, D, BATCH), jnp.float32),
    scratch_types=[
        pltpu.VMEM((BW, NF), jnp.int32),
        pltpu.VMEM((NF, IPITCH), jnp.int32),
        pltpu.VMEM((CH, D), jnp.float32),
        pltpu.VMEM((CH, D), jnp.float32),
        pltpu.VMEM((D, PITCH), jnp.float32),
        pltpu.VMEM((D, PITCH), jnp.float32),
        pltpu.SemaphoreType.DMA,
        pltpu.SemaphoreType.DMA,
        pltpu.SemaphoreType.DMA,
        pltpu.SemaphoreType.DMA,
    ],
    compiler_params=pltpu.CompilerParams(use_tc_tiling_on_sc=False,
                                         needs_layout_passes=False),
)
def _gather_rows(x_hbm, w_hbm, out_hbm, idx_raw, idx_t, rows0, rows1,
                 tr0, tr1, s_g0, s_g1, s_o0, s_o1):
    wid = lax.axis_index("s") * NC + lax.axis_index("c")
    b0 = wid * BW

    rows = (rows0, rows1)
    trs = (tr0, tr1)
    s_g = (s_g0, s_g1)
    s_o = (s_o0, s_o1)

    pltpu.sync_copy(x_hbm.at[pl.ds(b0, BW)], idx_raw)

    iota16 = lax.iota(jnp.int32, 16)

    def idx_body(j, carry):
        for jj in range(4):
            row = j * 4 + jj
            rvec = jnp.full((16,), row, jnp.int32)
            v_lo = idx_raw[row, pl.ds(0, 16)]
            v_hi = idx_raw[row, pl.ds(NF - 16, 16)]
            plsc.store_scatter(idx_t, [iota16, rvec], v_lo)
            plsc.store_scatter(idx_t, [iota16 + (NF - 16), rvec], v_hi)
        return carry

    lax.fori_loop(0, BW // 4, idx_body, 0)

    def split(i):
        f = i // 4
        tcl = lax.rem(i, 4)
        return f, tcl

    def issue_gather(i, par):
        f, tcl = split(i)
        return pltpu.async_copy(
            w_hbm.at[idx_t.at[f, pl.ds(tcl * CH, CH)]], rows[par], s_g[par])

    def issue_write(i, par):
        f, tcl = split(i)
        return pltpu.async_copy(
            trs[par].at[:, pl.ds(0, CH)],
            out_hbm.at[f, :, pl.ds(b0 + tcl * CH, CH)], s_o[par])

    def transpose(par):
        src, dst = rows[par], trs[par]

        def body_j(j, carry):
            for jj in range(8):
                row = j * 8 + jj
                rvec = jnp.full((16,), row, jnp.int32)
                v_lo = src[row, pl.ds(0, 16)]
                v_hi = src[row, pl.ds(16, 16)]
                plsc.store_scatter(dst, [iota16, rvec], v_lo)
                plsc.store_scatter(dst, [iota16 + 16, rvec], v_hi)
            return carry

        lax.fori_loop(0, CH // 8, body_j, 0)

    issue_gather(0, 0)
    issue_gather(1, 1)

    def step(k, carry):
        for par in range(2):
            i = 2 * k + par
            pltpu.make_async_copy(w_hbm.at[idx_t.at[0, pl.ds(0, CH)]],
                                  rows[par], s_g[par]).wait()

            @pl.when(i >= 2)
            def _():
                pltpu.make_async_copy(
                    trs[par].at[:, pl.ds(0, CH)],
                    out_hbm.at[0, :, pl.ds(0, CH)], s_o[par]).wait()

            transpose(par)
            issue_write(i, par)

            @pl.when(i + 2 < NCHUNK)
            def _():
                issue_gather(i + 2, par)
        return carry

    lax.fori_loop(0, NCHUNK // 2, step, 0)

    for par in range(2):
        pltpu.make_async_copy(trs[par].at[:, pl.ds(0, CH)],
                              out_hbm.at[0, :, pl.ds(0, CH)], s_o[par]).wait()


def kernel(x, weight):
    out = _gather_rows(x, weight)
    return out.transpose(2, 0, 1)


# speedup vs baseline: 1.0015x; 1.0015x over previous
import functools

import jax
import jax.numpy as jnp
from jax import lax
from jax.experimental import pallas as pl
from jax.experimental.pallas import tpu as pltpu
from jax.experimental.pallas import tpu_sc as plsc

D = 32
NF = 26
BATCH = 16384
NC, NS = 2, 16
NW = NC * NS
BW = BATCH // NW
CH = 128
NCHUNK = NF * (BW // CH)
PITCH = 129
IPITCH = 520

_mesh = plsc.VectorSubcoreMesh(core_axis_name="c", subcore_axis_name="s")


@functools.partial(
    pl.kernel,
    mesh=_mesh,
    out_type=jax.ShapeDtypeStruct((NF, D, BATCH), jnp.float32),
    scratch_types=[
        pltpu.VMEM((BW, NF), jnp.float32),
        pltpu.VMEM((NF, IPITCH), jnp.int32),
        pltpu.VMEM((CH, D), jnp.float32),
        pltpu.VMEM((CH, D), jnp.float32),
        pltpu.VMEM((D, PITCH), jnp.float32),
        pltpu.VMEM((D, PITCH), jnp.float32),
        pltpu.SemaphoreType.DMA,
        pltpu.SemaphoreType.DMA,
        pltpu.SemaphoreType.DMA,
        pltpu.SemaphoreType.DMA,
    ],
    compiler_params=pltpu.CompilerParams(use_tc_tiling_on_sc=False,
                                         needs_layout_passes=False),
)
def _gather_rows(x_hbm, w_hbm, out_hbm, idx_raw, idx_t, rows0, rows1,
                 tr0, tr1, s_g0, s_g1, s_o0, s_o1):
    wid = lax.axis_index("s") * NC + lax.axis_index("c")
    b0 = wid * BW

    rows = (rows0, rows1)
    trs = (tr0, tr1)
    s_g = (s_g0, s_g1)
    s_o = (s_o0, s_o1)

    pltpu.sync_copy(x_hbm.at[pl.ds(b0, BW)], idx_raw)

    iota16 = lax.iota(jnp.int32, 16)

    def idx_body(j, carry):
        for jj in range(4):
            row = j * 4 + jj
            rvec = jnp.full((16,), row, jnp.int32)
            v_lo = plsc.bitcast(idx_raw[row, pl.ds(0, 16)], jnp.int32)
            v_hi = plsc.bitcast(idx_raw[row, pl.ds(NF - 16, 16)], jnp.int32)
            plsc.store_scatter(idx_t, [iota16, rvec], v_lo)
            plsc.store_scatter(idx_t, [iota16 + (NF - 16), rvec], v_hi)
        return carry

    lax.fori_loop(0, BW // 4, idx_body, 0)

    def split(i):
        f = i // 4
        tcl = lax.rem(i, 4)
        return f, tcl

    def issue_gather(i, par):
        f, tcl = split(i)
        return pltpu.async_copy(
            w_hbm.at[idx_t.at[f, pl.ds(tcl * CH, CH)]], rows[par], s_g[par])

    def issue_write(i, par):
        f, tcl = split(i)
        return pltpu.async_copy(
            trs[par].at[:, pl.ds(0, CH)],
            out_hbm.at[f, :, pl.ds(b0 + tcl * CH, CH)], s_o[par])

    def transpose(par):
        src, dst = rows[par], trs[par]

        def body_j(j, carry):
            for jj in range(8):
                row = j * 8 + jj
                rvec = jnp.full((16,), row, jnp.int32)
                v_lo = src[row, pl.ds(0, 16)]
                v_hi = src[row, pl.ds(16, 16)]
                plsc.store_scatter(dst, [iota16, rvec], v_lo)
                plsc.store_scatter(dst, [iota16 + 16, rvec], v_hi)
            return carry

        lax.fori_loop(0, CH // 8, body_j, 0)

    issue_gather(0, 0)
    issue_gather(1, 1)

    def step(k, carry):
        for par in range(2):
            i = 2 * k + par
            pltpu.make_async_copy(w_hbm.at[idx_t.at[0, pl.ds(0, CH)]],
                                  rows[par], s_g[par]).wait()

            @pl.when(i >= 2)
            def _():
                pltpu.make_async_copy(
                    trs[par].at[:, pl.ds(0, CH)],
                    out_hbm.at[0, :, pl.ds(0, CH)], s_o[par]).wait()

            transpose(par)
            issue_write(i, par)

            @pl.when(i + 2 < NCHUNK)
            def _():
                issue_gather(i + 2, par)
        return carry

    lax.fori_loop(0, NCHUNK // 2, step, 0)

    for par in range(2):
        pltpu.make_async_copy(trs[par].at[:, pl.ds(0, CH)],
                              out_hbm.at[0, :, pl.ds(0, CH)], s_o[par]).wait()


def kernel(x, weight):
    xf = jax.lax.bitcast_convert_type(x, jnp.float32)
    out = _gather_rows(xf, weight)
    return out.transpose(2, 0, 1)
